# baseline (speedup 1.0000x reference)
import functools
import math

import jax
import jax.numpy as jnp
from jax import lax
from jax.experimental import pallas as pl
from jax.experimental.pallas import tpu as pltpu
from jax.experimental.pallas import tpu_sc as plsc

N = 10000
E = 160000
D = 256
H = 256
C = 64
L = 4
ALPHA = 0.1
THETA = 0.5

HH = H // 2
NT = 16
EPT = E // NT
CHUNK = 125
NCH = EPT // CHUNK
RPT = 640

BLK = 1000
GRID = N // BLK

_BETAS = [math.log(THETA / (i + 1) + 1.0) for i in range(L)]


def _mesh():
    return plsc.VectorSubcoreMesh(core_axis_name="c", subcore_axis_name="s")


def _sc_deg_body(col_hbm, deg_hbm, colv, ones_b, zbuf, hist):
    c = lax.axis_index("c")
    s = lax.axis_index("s")

    pltpu.sync_copy(col_hbm.at[s], colv)

    lane = lax.iota(jnp.int32, 16)
    pattern = jnp.where(lane == 0, 1.0, 0.0).astype(jnp.float32)
    zero16 = jnp.zeros((16,), jnp.float32)
    for r in range(CHUNK):
        for q in range(HH // 16):
            ones_b[r, pl.ds(q * 16, 16)] = pattern if q == 0 else zero16
    for r in range(40):
        for q in range(HH // 16):
            zbuf[r, pl.ds(q * 16, 16)] = zero16

    base = s * RPT
    nz = jnp.where(s == NT - 1, 10, 16)

    def zloop(z, carry):
        pltpu.sync_copy(zbuf, hist.at[pl.ds(base + z * 40, 40)])
        return carry

    lax.fori_loop(0, nz, zloop, 0)
    plsc.subcore_barrier()

    half = NCH // 2

    def body(j, carry):
        pltpu.sync_copy(ones_b, hist.at[colv.at[c * half + j]], add=True)
        return carry

    lax.fori_loop(0, half, body, 0)
    plsc.subcore_barrier()

    @pl.when(s < NT - 1)
    def _():
        pltpu.sync_copy(hist.at[pl.ds(base, RPT)],
                        deg_hbm.at[pl.ds(c * N + base, RPT)])

    @pl.when(s == NT - 1)
    def _():
        pltpu.sync_copy(hist.at[pl.ds((NT - 1) * RPT, N - (NT - 1) * RPT)],
                        deg_hbm.at[pl.ds(c * N + (NT - 1) * RPT,
                                         N - (NT - 1) * RPT)])


def _sc_deg(col3):
    kern = pl.kernel(
        _sc_deg_body,
        mesh=_mesh(),
        out_type=jax.ShapeDtypeStruct((2 * N, HH), jnp.float32),
        scratch_types=[
            pltpu.VMEM((NCH, CHUNK), jnp.int32),
            pltpu.VMEM((CHUNK, HH), jnp.float32),
            pltpu.VMEM((40, HH), jnp.float32),
            pltpu.VMEM_SHARED((N, HH), jnp.float32),
        ],
    )
    return kern(col3)


NPH = 2
CPP = NCH // NPH
GCH = 128


def _sc_scatter_body(tplo_hbm, tphi_hbm, row_hbm, col_hbm,
                     agglo_hbm, agghi_hbm,
                     rowv, colv, buf0, buf1, aggs, sem0, sem1):
    c = lax.axis_index("c")
    s = lax.axis_index("s")

    base = s * RPT

    def run(tp_hbm, agg_hbm):
        pltpu.sync_copy(row_hbm.at[s, pl.ds(0, CPP)], rowv)
        pltpu.sync_copy(col_hbm.at[s, pl.ds(0, CPP)], colv)
        pltpu.make_async_copy(tp_hbm.at[rowv.at[0]],
                              buf0.at[pl.ds(0, CHUNK)], sem0).start()

        zero16 = jnp.zeros((16,), jnp.float32)
        for r in range(GCH):
            for q in range(HH // 16):
                buf1[r, pl.ds(q * 16, 16)] = zero16

        nz = jnp.where(s == NT - 1, 3, 5)

        def zloop(z, carry):
            pltpu.sync_copy(buf1, aggs.at[pl.ds(base + z * GCH, GCH)])
            return carry

        lax.fori_loop(0, nz, zloop, 0)

        @pl.when(s == NT - 1)
        def _():
            pltpu.sync_copy(buf1.at[pl.ds(0, 16)],
                            aggs.at[pl.ds((NT - 1) * RPT + 3 * GCH, 16)])

        plsc.subcore_barrier()

        for p in range(NPH):
            if p > 0:
                pltpu.sync_copy(row_hbm.at[s, pl.ds(p * CPP, CPP)], rowv)
                pltpu.sync_copy(col_hbm.at[s, pl.ds(p * CPP, CPP)], colv)
                pltpu.make_async_copy(tp_hbm.at[rowv.at[0]],
                                      buf0.at[pl.ds(0, CHUNK)], sem0).start()

            def body(i, carry):
                j0 = 2 * i
                j1 = 2 * i + 1
                b0 = buf0.at[pl.ds(0, CHUNK)]
                b1 = buf1.at[pl.ds(0, CHUNK)]
                pltpu.make_async_copy(tp_hbm.at[rowv.at[j0]], b0,
                                      sem0).wait()
                pltpu.make_async_copy(tp_hbm.at[rowv.at[j1]], b1,
                                      sem1).start()
                pltpu.sync_copy(b0, aggs.at[colv.at[j0]], add=True)
                pltpu.make_async_copy(tp_hbm.at[rowv.at[j1]], b1,
                                      sem1).wait()

                @pl.when(i < CPP // 2 - 1)
                def _():
                    pltpu.make_async_copy(tp_hbm.at[rowv.at[j0 + 2]], b0,
                                          sem0).start()

                pltpu.sync_copy(b1, aggs.at[colv.at[j1]], add=True)
                return carry

            lax.fori_loop(0, CPP // 2, body, 0)
        plsc.subcore_barrier()

        @pl.when(s < NT - 1)
        def _():
            pltpu.sync_copy(aggs.at[pl.ds(base, RPT)],
                            agg_hbm.at[pl.ds(base, RPT)])

        @pl.when(s == NT - 1)
        def _():
            tail = N - (NT - 1) * RPT
            pltpu.sync_copy(aggs.at[pl.ds((NT - 1) * RPT, tail)],
                            agg_hbm.at[pl.ds((NT - 1) * RPT, tail)])

    @pl.when(c == 0)
    def _():
        run(tplo_hbm, agglo_hbm)

    @pl.when(c == 1)
    def _():
        run(tphi_hbm, agghi_hbm)


def _sc_scatter(tplo, tphi, row3, col3):
    kern = pl.kernel(
        _sc_scatter_body,
        mesh=_mesh(),
        out_type=[jax.ShapeDtypeStruct((N, HH), jnp.float32),
                  jax.ShapeDtypeStruct((N, HH), jnp.float32)],
        scratch_types=[
            pltpu.VMEM((CPP, CHUNK), jnp.int32),
            pltpu.VMEM((CPP, CHUNK), jnp.int32),
            pltpu.VMEM((GCH, HH), jnp.float32),
            pltpu.VMEM((GCH, HH), jnp.float32),
            pltpu.VMEM_SHARED((N, HH), jnp.float32),
            pltpu.SemaphoreType.DMA,
            pltpu.SemaphoreType.DMA,
        ],
    )
    return kern(tplo, tphi, row3, col3)


def _row_spec(width):
    return pl.BlockSpec((BLK, width), lambda i: (i, 0))


def _full_spec(shape):
    nd = len(shape)
    return pl.BlockSpec(shape, lambda i: (0,) * nd)


def _deg_specs():
    lo = pl.BlockSpec((BLK, HH), lambda i: (i, 0))
    hi = pl.BlockSpec((BLK, HH), lambda i: (i + GRID, 0))
    return lo, hi


def _dis_from(dlo, dhi):
    deg = 1.0 + dlo[:, 0:1] + dhi[:, 0:1]
    return lax.rsqrt(deg)


def _k0a_body(x_ref, w0t_ref, b0_ref, x0_ref):
    h = jnp.dot(x_ref[...], w0t_ref[...], preferred_element_type=jnp.float32)
    x0_ref[...] = jnp.maximum(h + b0_ref[...], 0.0)


def _k0a(x, fc0_wt, fc0_b2):
    return pl.pallas_call(
        _k0a_body,
        grid=(GRID,),
        in_specs=[_row_spec(D), _full_spec((D, H)), _full_spec((1, H))],
        out_specs=[_row_spec(H)],
        out_shape=[jax.ShapeDtypeStruct((N, H), jnp.float32)],
    )(x, fc0_wt, fc0_b2)[0]


def _ku_body(ci, bi, x0_ref, w2_ref, u_ref):
    x0 = x0_ref[...]
    u_ref[...] = ci * x0 + bi * jnp.dot(x0, w2_ref[...],
                                        preferred_element_type=jnp.float32)


def _ku(i, x0, w2_i):
    bi = _BETAS[i]
    body = functools.partial(_ku_body, (1.0 - bi) * ALPHA, bi)
    return pl.pallas_call(
        body,
        grid=(GRID,),
        in_specs=[_row_spec(H), _full_spec((H, H))],
        out_specs=[_row_spec(H)],
        out_shape=[jax.ShapeDtypeStruct((N, H), jnp.float32)],
    )(x0, w2_i)[0]


def _k0b_body(a0, b0, h_ref, w1_ref, dlo_ref, dhi_ref,
              tplo_ref, tphi_ref, dis_ref):
    h = h_ref[...]
    dis = _dis_from(dlo_ref[...], dhi_ref[...])
    dis_ref[...] = jnp.broadcast_to(dis, (BLK, HH))
    t = a0 * h + b0 * jnp.dot(h, w1_ref[...],
                              preferred_element_type=jnp.float32)
    tp = dis * t
    tplo_ref[...] = tp[:, :HH]
    tphi_ref[...] = tp[:, HH:]


def _k0b(x0, w1_0, deg):
    beta = _BETAS[0]
    body = functools.partial(_k0b_body, (1.0 - beta) * (1.0 - ALPHA), beta)
    dlo, dhi = _deg_specs()
    return pl.pallas_call(
        body,
        grid=(GRID,),
        in_specs=[_row_spec(H), _full_spec((H, H)), dlo, dhi],
        out_specs=[_row_spec(HH), _row_spec(HH), _row_spec(HH)],
        out_shape=[jax.ShapeDtypeStruct((N, HH), jnp.float32),
                   jax.ShapeDtypeStruct((N, HH), jnp.float32),
                   jax.ShapeDtypeStruct((N, HH), jnp.float32)],
    )(x0, w1_0, deg, deg)


def _kcomb_body(an, bn, agglo_ref, agghi_ref, tplo_ref, tphi_ref,
                u_ref, w1_ref, dis_ref,
                otplo_ref, otphi_ref):
    dis = dis_ref[...][:, 0:1]
    agg = jnp.concatenate([agglo_ref[...], agghi_ref[...]], axis=1)
    tp = jnp.concatenate([tplo_ref[...], tphi_ref[...]], axis=1)
    h = jnp.maximum(dis * (agg + tp) + u_ref[...], 0.0)
    t = an * h + bn * jnp.dot(h, w1_ref[...],
                              preferred_element_type=jnp.float32)
    tp_new = dis * t
    otplo_ref[...] = tp_new[:, :HH]
    otphi_ref[...] = tp_new[:, HH:]


def _kcomb(i, agglo, agghi, tplo, tphi, u_i, w1_n, dis):
    bn = _BETAS[i + 1]
    body = functools.partial(_kcomb_body, (1.0 - bn) * (1.0 - ALPHA), bn)
    return pl.pallas_call(
        body,
        grid=(GRID,),
        in_specs=[_row_spec(HH), _row_spec(HH), _row_spec(HH), _row_spec(HH),
                  _row_spec(H), _full_spec((H, H)), _row_spec(HH)],
        out_specs=[_row_spec(HH), _row_spec(HH)],
        out_shape=[jax.ShapeDtypeStruct((N, HH), jnp.float32),
                   jax.ShapeDtypeStruct((N, HH), jnp.float32)],
    )(agglo, agghi, tplo, tphi, u_i, w1_n, dis)


def _kfin_body(agglo_ref, agghi_ref, tplo_ref, tphi_ref,
               u_ref, fc1t_ref, fc1b_ref, dis_ref, y_ref):
    dis = dis_ref[...][:, 0:1]
    agg = jnp.concatenate([agglo_ref[...], agghi_ref[...]], axis=1)
    tp = jnp.concatenate([tplo_ref[...], tphi_ref[...]], axis=1)
    h = jnp.maximum(dis * (agg + tp) + u_ref[...], 0.0)
    y_ref[...] = jnp.dot(h, fc1t_ref[...],
                         preferred_element_type=jnp.float32) + fc1b_ref[...]


def _kfin(agglo, agghi, tplo, tphi, u_i, fc1_wt, fc1_b2, dis):
    return pl.pallas_call(
        _kfin_body,
        grid=(GRID,),
        in_specs=[_row_spec(HH), _row_spec(HH), _row_spec(HH), _row_spec(HH),
                  _row_spec(H), _full_spec((H, C)), _full_spec((1, C)),
                  _row_spec(HH)],
        out_specs=[_row_spec(C)],
        out_shape=[jax.ShapeDtypeStruct((N, C), jnp.float32)],
    )(agglo, agghi, tplo, tphi, u_i, fc1_wt, fc1_b2, dis)[0]


def kernel(x, edge_index, fc0_w, fc0_b, fc1_w, fc1_b, w1, w2):
    row3 = edge_index[0].reshape(NT, NCH, CHUNK)
    col3 = edge_index[1].reshape(NT, NCH, CHUNK)

    deg = _sc_deg(col3)
    x0 = _k0a(x, fc0_w.T, fc0_b.reshape(1, H))

    u = [_ku(i, x0, w2[i]) for i in range(L)]

    tplo, tphi, dis = _k0b(x0, w1[0], deg)
    for i in range(L - 1):
        agglo, agghi = _sc_scatter(tplo, tphi, row3, col3)
        tplo, tphi = _kcomb(i, agglo, agghi, tplo, tphi, u[i],
                            w1[i + 1], dis)
    agglo, agghi = _sc_scatter(tplo, tphi, row3, col3)
    return _kfin(agglo, agghi, tplo, tphi, u[L - 1],
                 fc1_w.T, fc1_b.reshape(1, C), dis)


# speedup vs baseline: 1.1423x; 1.1423x over previous
import functools
import math

import jax
import jax.numpy as jnp
from jax import lax
from jax.experimental import pallas as pl
from jax.experimental.pallas import tpu as pltpu
from jax.experimental.pallas import tpu_sc as plsc

N = 10000
E = 160000
D = 256
H = 256
C = 64
L = 4
ALPHA = 0.1
THETA = 0.5

HH = H // 2
NT = 16
EPT = E // NT
CHUNK = 80
NCH = EPT // CHUNK
DCH = 100
DCPP = 50
RPT = 640

BLK = 1000
GRID = N // BLK

_BETAS = [math.log(THETA / (i + 1) + 1.0) for i in range(L)]


def _mesh():
    return plsc.VectorSubcoreMesh(core_axis_name="c", subcore_axis_name="s")


def _sc_deg_body(col_hbm, deg_hbm, colv, ones_b, zbuf, hist):
    c = lax.axis_index("c")
    s = lax.axis_index("s")

    pltpu.sync_copy(col_hbm.at[s * 2 + c], colv)

    lane = lax.iota(jnp.int32, 16)
    pattern = jnp.where(lane == 0, 1.0, 0.0).astype(jnp.float32)
    zero16 = jnp.zeros((16,), jnp.float32)
    for r in range(DCH):
        for q in range(HH // 16):
            ones_b[r, pl.ds(q * 16, 16)] = pattern if q == 0 else zero16
    for r in range(40):
        for q in range(HH // 16):
            zbuf[r, pl.ds(q * 16, 16)] = zero16

    base = s * RPT
    nz = jnp.where(s == NT - 1, 10, 16)

    def zloop(z, carry):
        pltpu.sync_copy(zbuf, hist.at[pl.ds(base + z * 40, 40)])
        return carry

    lax.fori_loop(0, nz, zloop, 0)
    plsc.subcore_barrier()

    def body(j, carry):
        pltpu.sync_copy(ones_b, hist.at[colv.at[j]], add=True)
        return carry

    lax.fori_loop(0, DCPP, body, 0)
    plsc.subcore_barrier()

    @pl.when(s < NT - 1)
    def _():
        pltpu.sync_copy(hist.at[pl.ds(base, RPT)],
                        deg_hbm.at[pl.ds(c * N + base, RPT)])

    @pl.when(s == NT - 1)
    def _():
        pltpu.sync_copy(hist.at[pl.ds((NT - 1) * RPT, N - (NT - 1) * RPT)],
                        deg_hbm.at[pl.ds(c * N + (NT - 1) * RPT,
                                         N - (NT - 1) * RPT)])


def _sc_deg(col3):
    kern = pl.kernel(
        _sc_deg_body,
        mesh=_mesh(),
        out_type=jax.ShapeDtypeStruct((2 * N, HH), jnp.float32),
        scratch_types=[
            pltpu.VMEM((DCPP, DCH), jnp.int32),
            pltpu.VMEM((DCH, HH), jnp.float32),
            pltpu.VMEM((40, HH), jnp.float32),
            pltpu.VMEM_SHARED((N, HH), jnp.float32),
        ],
    )
    return kern(col3)


NPH = 5
CPP = NCH // NPH


def _sc_scatter_body(tplo_hbm, tphi_hbm, row_hbm, col_hbm,
                     agglo_hbm, agghi_hbm,
                     rowv, colv, buf0, buf1, buf2, aggs, sem0, sem1, sem2):
    c = lax.axis_index("c")
    s = lax.axis_index("s")

    base = s * RPT

    def run(tp_hbm, agg_hbm):
        pltpu.sync_copy(row_hbm.at[s * NPH], rowv)
        pltpu.sync_copy(col_hbm.at[s * NPH], colv)
        pltpu.make_async_copy(tp_hbm.at[rowv.at[0]], buf0, sem0).start()
        pltpu.make_async_copy(tp_hbm.at[rowv.at[1]], buf1, sem1).start()

        zero16 = jnp.zeros((16,), jnp.float32)
        for r in range(CHUNK):
            for q in range(HH // 16):
                buf2[r, pl.ds(q * 16, 16)] = zero16

        nz = jnp.where(s == NT - 1, 5, 8)

        def zloop(z, carry):
            pltpu.sync_copy(buf2, aggs.at[pl.ds(base + z * CHUNK, CHUNK)])
            return carry

        lax.fori_loop(0, nz, zloop, 0)
        plsc.subcore_barrier()

        bufs = (buf0, buf1, buf2)
        sems = (sem0, sem1, sem2)

        for p in range(NPH):
            if p > 0:
                pltpu.sync_copy(row_hbm.at[s * NPH + p], rowv)
                pltpu.sync_copy(col_hbm.at[s * NPH + p], colv)
                pltpu.make_async_copy(tp_hbm.at[rowv.at[0]], buf0,
                                      sem0).start()
                pltpu.make_async_copy(tp_hbm.at[rowv.at[1]], buf1,
                                      sem1).start()

            def body(i, carry):
                for k in range(3):
                    j = 3 * i + k
                    b, sem = bufs[k], sems[k]
                    bn, semn = bufs[(k + 2) % 3], sems[(k + 2) % 3]
                    pltpu.make_async_copy(tp_hbm.at[rowv.at[j]], b,
                                          sem).wait()
                    pltpu.make_async_copy(tp_hbm.at[rowv.at[j + 2]], bn,
                                          semn).start()
                    pltpu.sync_copy(b, aggs.at[colv.at[j]], add=True)
                return carry

            nb = (CPP - 2) // 3
            lax.fori_loop(0, nb, body, 0)
            for j in range(3 * nb, CPP):
                b, sem = bufs[j % 3], sems[j % 3]
                pltpu.make_async_copy(tp_hbm.at[rowv.at[j]], b, sem).wait()
                if j + 2 < CPP:
                    bn, semn = bufs[(j + 2) % 3], sems[(j + 2) % 3]
                    pltpu.make_async_copy(tp_hbm.at[rowv.at[j + 2]], bn,
                                          semn).start()
                pltpu.sync_copy(b, aggs.at[colv.at[j]], add=True)
        plsc.subcore_barrier()

        @pl.when(s < NT - 1)
        def _():
            pltpu.sync_copy(aggs.at[pl.ds(base, RPT)],
                            agg_hbm.at[pl.ds(base, RPT)])

        @pl.when(s == NT - 1)
        def _():
            tail = N - (NT - 1) * RPT
            pltpu.sync_copy(aggs.at[pl.ds((NT - 1) * RPT, tail)],
                            agg_hbm.at[pl.ds((NT - 1) * RPT, tail)])

    @pl.when(c == 0)
    def _():
        run(tplo_hbm, agglo_hbm)

    @pl.when(c == 1)
    def _():
        run(tphi_hbm, agghi_hbm)


def _sc_scatter(tplo, tphi, row3, col3):
    kern = pl.kernel(
        _sc_scatter_body,
        mesh=_mesh(),
        out_type=[jax.ShapeDtypeStruct((N, HH), jnp.float32),
                  jax.ShapeDtypeStruct((N, HH), jnp.float32)],
        scratch_types=[
            pltpu.VMEM((CPP, CHUNK), jnp.int32),
            pltpu.VMEM((CPP, CHUNK), jnp.int32),
            pltpu.VMEM((CHUNK, HH), jnp.float32),
            pltpu.VMEM((CHUNK, HH), jnp.float32),
            pltpu.VMEM((CHUNK, HH), jnp.float32),
            pltpu.VMEM_SHARED((N, HH), jnp.float32),
            pltpu.SemaphoreType.DMA,
            pltpu.SemaphoreType.DMA,
            pltpu.SemaphoreType.DMA,
        ],
    )
    return kern(tplo, tphi, row3, col3)


def _row_spec(width):
    return pl.BlockSpec((BLK, width), lambda i: (i, 0))


def _full_spec(shape):
    nd = len(shape)
    return pl.BlockSpec(shape, lambda i: (0,) * nd)


def _deg_specs():
    lo = pl.BlockSpec((BLK, HH), lambda i: (i, 0))
    hi = pl.BlockSpec((BLK, HH), lambda i: (i + GRID, 0))
    return lo, hi


def _dis_from(dlo, dhi):
    deg = 1.0 + dlo[:, 0:1] + dhi[:, 0:1]
    return lax.rsqrt(deg)


def _k0a_body(x_ref, w0t_ref, b0_ref, x0_ref):
    h = jnp.dot(x_ref[...], w0t_ref[...], preferred_element_type=jnp.float32)
    x0_ref[...] = jnp.maximum(h + b0_ref[...], 0.0)


def _k0a(x, fc0_wt, fc0_b2):
    return pl.pallas_call(
        _k0a_body,
        grid=(GRID,),
        in_specs=[_row_spec(D), _full_spec((D, H)), _full_spec((1, H))],
        out_specs=[_row_spec(H)],
        out_shape=[jax.ShapeDtypeStruct((N, H), jnp.float32)],
    )(x, fc0_wt, fc0_b2)[0]


def _ku_body(ci, bi, x0_ref, w2_ref, u_ref):
    x0 = x0_ref[...]
    u_ref[...] = ci * x0 + bi * jnp.dot(x0, w2_ref[...],
                                        preferred_element_type=jnp.float32)


def _ku(i, x0, w2_i):
    bi = _BETAS[i]
    body = functools.partial(_ku_body, (1.0 - bi) * ALPHA, bi)
    return pl.pallas_call(
        body,
        grid=(GRID,),
        in_specs=[_row_spec(H), _full_spec((H, H))],
        out_specs=[_row_spec(H)],
        out_shape=[jax.ShapeDtypeStruct((N, H), jnp.float32)],
    )(x0, w2_i)[0]


def _k0b_body(a0, b0, h_ref, w1_ref, dlo_ref, dhi_ref,
              tplo_ref, tphi_ref, dis_ref):
    h = h_ref[...]
    dis = _dis_from(dlo_ref[...], dhi_ref[...])
    dis_ref[...] = jnp.broadcast_to(dis, (BLK, HH))
    t = a0 * h + b0 * jnp.dot(h, w1_ref[...],
                              preferred_element_type=jnp.float32)
    tp = dis * t
    tplo_ref[...] = tp[:, :HH]
    tphi_ref[...] = tp[:, HH:]


def _k0b(x0, w1_0, deg):
    beta = _BETAS[0]
    body = functools.partial(_k0b_body, (1.0 - beta) * (1.0 - ALPHA), beta)
    dlo, dhi = _deg_specs()
    return pl.pallas_call(
        body,
        grid=(GRID,),
        in_specs=[_row_spec(H), _full_spec((H, H)), dlo, dhi],
        out_specs=[_row_spec(HH), _row_spec(HH), _row_spec(HH)],
        out_shape=[jax.ShapeDtypeStruct((N, HH), jnp.float32),
                   jax.ShapeDtypeStruct((N, HH), jnp.float32),
                   jax.ShapeDtypeStruct((N, HH), jnp.float32)],
    )(x0, w1_0, deg, deg)


def _kcomb_body(an, bn, agglo_ref, agghi_ref, tplo_ref, tphi_ref,
                u_ref, w1_ref, dis_ref,
                otplo_ref, otphi_ref):
    dis = dis_ref[...][:, 0:1]
    agg = jnp.concatenate([agglo_ref[...], agghi_ref[...]], axis=1)
    tp = jnp.concatenate([tplo_ref[...], tphi_ref[...]], axis=1)
    h = jnp.maximum(dis * (agg + tp) + u_ref[...], 0.0)
    t = an * h + bn * jnp.dot(h, w1_ref[...],
                              preferred_element_type=jnp.float32)
    tp_new = dis * t
    otplo_ref[...] = tp_new[:, :HH]
    otphi_ref[...] = tp_new[:, HH:]


def _kcomb(i, agglo, agghi, tplo, tphi, u_i, w1_n, dis):
    bn = _BETAS[i + 1]
    body = functools.partial(_kcomb_body, (1.0 - bn) * (1.0 - ALPHA), bn)
    return pl.pallas_call(
        body,
        grid=(GRID,),
        in_specs=[_row_spec(HH), _row_spec(HH), _row_spec(HH), _row_spec(HH),
                  _row_spec(H), _full_spec((H, H)), _row_spec(HH)],
        out_specs=[_row_spec(HH), _row_spec(HH)],
        out_shape=[jax.ShapeDtypeStruct((N, HH), jnp.float32),
                   jax.ShapeDtypeStruct((N, HH), jnp.float32)],
    )(agglo, agghi, tplo, tphi, u_i, w1_n, dis)


def _kfin_body(agglo_ref, agghi_ref, tplo_ref, tphi_ref,
               u_ref, fc1t_ref, fc1b_ref, dis_ref, y_ref):
    dis = dis_ref[...][:, 0:1]
    agg = jnp.concatenate([agglo_ref[...], agghi_ref[...]], axis=1)
    tp = jnp.concatenate([tplo_ref[...], tphi_ref[...]], axis=1)
    h = jnp.maximum(dis * (agg + tp) + u_ref[...], 0.0)
    y_ref[...] = jnp.dot(h, fc1t_ref[...],
                         preferred_element_type=jnp.float32) + fc1b_ref[...]


def _kfin(agglo, agghi, tplo, tphi, u_i, fc1_wt, fc1_b2, dis):
    return pl.pallas_call(
        _kfin_body,
        grid=(GRID,),
        in_specs=[_row_spec(HH), _row_spec(HH), _row_spec(HH), _row_spec(HH),
                  _row_spec(H), _full_spec((H, C)), _full_spec((1, C)),
                  _row_spec(HH)],
        out_specs=[_row_spec(C)],
        out_shape=[jax.ShapeDtypeStruct((N, C), jnp.float32)],
    )(agglo, agghi, tplo, tphi, u_i, fc1_wt, fc1_b2, dis)[0]


def kernel(x, edge_index, fc0_w, fc0_b, fc1_w, fc1_b, w1, w2):
    row3 = edge_index[0].reshape(NT * NPH, CPP, CHUNK)
    col3 = edge_index[1].reshape(NT * NPH, CPP, CHUNK)
    col_deg = edge_index[1].reshape(NT * 2, DCPP, DCH)

    deg = _sc_deg(col_deg)
    x0 = _k0a(x, fc0_w.T, fc0_b.reshape(1, H))

    u = [_ku(i, x0, w2[i]) for i in range(L)]

    tplo, tphi, dis = _k0b(x0, w1[0], deg)
    for i in range(L - 1):
        agglo, agghi = _sc_scatter(tplo, tphi, row3, col3)
        tplo, tphi = _kcomb(i, agglo, agghi, tplo, tphi, u[i],
                            w1[i + 1], dis)
    agglo, agghi = _sc_scatter(tplo, tphi, row3, col3)
    return _kfin(agglo, agghi, tplo, tphi, u[L - 1],
                 fc1_w.T, fc1_b.reshape(1, C), dis)


# speedup vs baseline: 1.1787x; 1.0318x over previous
import functools
import math

import jax
import jax.numpy as jnp
from jax import lax
from jax.experimental import pallas as pl
from jax.experimental.pallas import tpu as pltpu
from jax.experimental.pallas import tpu_sc as plsc

N = 10000
E = 160000
D = 256
H = 256
C = 64
L = 4
ALPHA = 0.1
THETA = 0.5

HH = H // 2
NT = 16
EPT = E // NT
CHUNK = 80
NCH = EPT // CHUNK
DCH = 100
DCPP = 50
RPT = 640

BLK = 1000
GRID = N // BLK

_BETAS = [math.log(THETA / (i + 1) + 1.0) for i in range(L)]


def _mesh():
    return plsc.VectorSubcoreMesh(core_axis_name="c", subcore_axis_name="s")


def _sc_deg_body(col_hbm, deg_hbm, colv, ones_b, zbuf, hist):
    c = lax.axis_index("c")
    s = lax.axis_index("s")

    pltpu.sync_copy(col_hbm.at[s * 2 + c], colv)

    lane = lax.iota(jnp.int32, 16)
    pattern = jnp.where(lane == 0, 1.0, 0.0).astype(jnp.float32)
    zero16 = jnp.zeros((16,), jnp.float32)
    for r in range(DCH):
        for q in range(HH // 16):
            ones_b[r, pl.ds(q * 16, 16)] = pattern if q == 0 else zero16
    for r in range(40):
        for q in range(HH // 16):
            zbuf[r, pl.ds(q * 16, 16)] = zero16

    base = s * RPT
    nz = jnp.where(s == NT - 1, 10, 16)

    def zloop(z, carry):
        pltpu.sync_copy(zbuf, hist.at[pl.ds(base + z * 40, 40)])
        return carry

    lax.fori_loop(0, nz, zloop, 0)
    plsc.subcore_barrier()

    def body(j, carry):
        pltpu.sync_copy(ones_b, hist.at[colv.at[j]], add=True)
        return carry

    lax.fori_loop(0, DCPP, body, 0)
    plsc.subcore_barrier()

    @pl.when(s < NT - 1)
    def _():
        pltpu.sync_copy(hist.at[pl.ds(base, RPT)],
                        deg_hbm.at[pl.ds(c * N + base, RPT)])

    @pl.when(s == NT - 1)
    def _():
        pltpu.sync_copy(hist.at[pl.ds((NT - 1) * RPT, N - (NT - 1) * RPT)],
                        deg_hbm.at[pl.ds(c * N + (NT - 1) * RPT,
                                         N - (NT - 1) * RPT)])


def _sc_deg(col3):
    kern = pl.kernel(
        _sc_deg_body,
        mesh=_mesh(),
        out_type=jax.ShapeDtypeStruct((2 * N, HH), jnp.float32),
        scratch_types=[
            pltpu.VMEM((DCPP, DCH), jnp.int32),
            pltpu.VMEM((DCH, HH), jnp.float32),
            pltpu.VMEM((40, HH), jnp.float32),
            pltpu.VMEM_SHARED((N, HH), jnp.float32),
        ],
    )
    return kern(col3)


NPH = 5
CPP = NCH // NPH


NBUF = 4


def _sc_scatter_body(tplo_hbm, tphi_hbm, row_hbm, col_hbm,
                     agglo_hbm, agghi_hbm,
                     rowv, colv, buf0, buf1, buf2, buf3, aggs,
                     sem0, sem1, sem2, sem3):
    c = lax.axis_index("c")
    s = lax.axis_index("s")

    base = s * RPT
    bufs = (buf0, buf1, buf2, buf3)
    sems = (sem0, sem1, sem2, sem3)
    NPRE = NBUF - 1

    def run(tp_hbm, agg_hbm):
        pltpu.sync_copy(row_hbm.at[s * NPH], rowv)
        pltpu.sync_copy(col_hbm.at[s * NPH], colv)
        for k in range(NPRE):
            pltpu.make_async_copy(tp_hbm.at[rowv.at[k]], bufs[k],
                                  sems[k]).start()

        zb = bufs[NBUF - 1]
        zero16 = jnp.zeros((16,), jnp.float32)
        for r in range(CHUNK):
            for q in range(HH // 16):
                zb[r, pl.ds(q * 16, 16)] = zero16

        nz = jnp.where(s == NT - 1, 5, 8)

        def zloop(z, carry):
            pltpu.sync_copy(zb, aggs.at[pl.ds(base + z * CHUNK, CHUNK)])
            return carry

        lax.fori_loop(0, nz, zloop, 0)
        plsc.subcore_barrier()

        for p in range(NPH):
            if p > 0:
                pltpu.sync_copy(row_hbm.at[s * NPH + p], rowv)
                pltpu.sync_copy(col_hbm.at[s * NPH + p], colv)
                for k in range(NPRE):
                    pltpu.make_async_copy(tp_hbm.at[rowv.at[k]], bufs[k],
                                          sems[k]).start()

            def body(i, carry):
                for k in range(NBUF):
                    j = NBUF * i + k
                    b, sem = bufs[k], sems[k]
                    kn = (k + NPRE) % NBUF
                    pltpu.make_async_copy(tp_hbm.at[rowv.at[j]], b,
                                          sem).wait()
                    pltpu.make_async_copy(tp_hbm.at[rowv.at[j + NPRE]],
                                          bufs[kn], sems[kn]).start()
                    pltpu.sync_copy(b, aggs.at[colv.at[j]], add=True)
                return carry

            nb = (CPP - NPRE) // NBUF
            lax.fori_loop(0, nb, body, 0)
            for j in range(NBUF * nb, CPP):
                b, sem = bufs[j % NBUF], sems[j % NBUF]
                pltpu.make_async_copy(tp_hbm.at[rowv.at[j]], b, sem).wait()
                if j + NPRE < CPP:
                    kn = (j + NPRE) % NBUF
                    pltpu.make_async_copy(tp_hbm.at[rowv.at[j + NPRE]],
                                          bufs[kn], sems[kn]).start()
                pltpu.sync_copy(b, aggs.at[colv.at[j]], add=True)
        plsc.subcore_barrier()

        @pl.when(s < NT - 1)
        def _():
            pltpu.sync_copy(aggs.at[pl.ds(base, RPT)],
                            agg_hbm.at[pl.ds(base, RPT)])

        @pl.when(s == NT - 1)
        def _():
            tail = N - (NT - 1) * RPT
            pltpu.sync_copy(aggs.at[pl.ds((NT - 1) * RPT, tail)],
                            agg_hbm.at[pl.ds((NT - 1) * RPT, tail)])

    @pl.when(c == 0)
    def _():
        run(tplo_hbm, agglo_hbm)

    @pl.when(c == 1)
    def _():
        run(tphi_hbm, agghi_hbm)


def _sc_scatter(tplo, tphi, row3, col3):
    kern = pl.kernel(
        _sc_scatter_body,
        mesh=_mesh(),
        out_type=[jax.ShapeDtypeStruct((N, HH), jnp.float32),
                  jax.ShapeDtypeStruct((N, HH), jnp.float32)],
        scratch_types=[
            pltpu.VMEM((CPP, CHUNK), jnp.int32),
            pltpu.VMEM((CPP, CHUNK), jnp.int32),
            pltpu.VMEM((CHUNK, HH), jnp.float32),
            pltpu.VMEM((CHUNK, HH), jnp.float32),
            pltpu.VMEM((CHUNK, HH), jnp.float32),
            pltpu.VMEM((CHUNK, HH), jnp.float32),
            pltpu.VMEM_SHARED((N, HH), jnp.float32),
            pltpu.SemaphoreType.DMA,
            pltpu.SemaphoreType.DMA,
            pltpu.SemaphoreType.DMA,
            pltpu.SemaphoreType.DMA,
        ],
    )
    return kern(tplo, tphi, row3, col3)


def _row_spec(width):
    return pl.BlockSpec((BLK, width), lambda i: (i, 0))


def _full_spec(shape):
    nd = len(shape)
    return pl.BlockSpec(shape, lambda i: (0,) * nd)


def _deg_specs():
    lo = pl.BlockSpec((BLK, HH), lambda i: (i, 0))
    hi = pl.BlockSpec((BLK, HH), lambda i: (i + GRID, 0))
    return lo, hi


def _dis_from(dlo, dhi):
    deg = 1.0 + dlo[:, 0:1] + dhi[:, 0:1]
    return lax.rsqrt(deg)


def _k0a_body(x_ref, w0t_ref, b0_ref, x0_ref):
    h = jnp.dot(x_ref[...], w0t_ref[...], preferred_element_type=jnp.float32)
    x0_ref[...] = jnp.maximum(h + b0_ref[...], 0.0)


def _k0a(x, fc0_wt, fc0_b2):
    return pl.pallas_call(
        _k0a_body,
        grid=(GRID,),
        in_specs=[_row_spec(D), _full_spec((D, H)), _full_spec((1, H))],
        out_specs=[_row_spec(H)],
        out_shape=[jax.ShapeDtypeStruct((N, H), jnp.float32)],
    )(x, fc0_wt, fc0_b2)[0]


def _ku_body(ci, bi, x0_ref, w2_ref, u_ref):
    x0 = x0_ref[...]
    u_ref[...] = ci * x0 + bi * jnp.dot(x0, w2_ref[...],
                                        preferred_element_type=jnp.float32)


def _ku(i, x0, w2_i):
    bi = _BETAS[i]
    body = functools.partial(_ku_body, (1.0 - bi) * ALPHA, bi)
    return pl.pallas_call(
        body,
        grid=(GRID,),
        in_specs=[_row_spec(H), _full_spec((H, H))],
        out_specs=[_row_spec(H)],
        out_shape=[jax.ShapeDtypeStruct((N, H), jnp.float32)],
    )(x0, w2_i)[0]


def _k0b_body(a0, b0, h_ref, w1_ref, dlo_ref, dhi_ref,
              tplo_ref, tphi_ref, dis_ref):
    h = h_ref[...]
    dis = _dis_from(dlo_ref[...], dhi_ref[...])
    dis_ref[...] = jnp.broadcast_to(dis, (BLK, HH))
    t = a0 * h + b0 * jnp.dot(h, w1_ref[...],
                              preferred_element_type=jnp.float32)
    tp = dis * t
    tplo_ref[...] = tp[:, :HH]
    tphi_ref[...] = tp[:, HH:]


def _k0b(x0, w1_0, deg):
    beta = _BETAS[0]
    body = functools.partial(_k0b_body, (1.0 - beta) * (1.0 - ALPHA), beta)
    dlo, dhi = _deg_specs()
    return pl.pallas_call(
        body,
        grid=(GRID,),
        in_specs=[_row_spec(H), _full_spec((H, H)), dlo, dhi],
        out_specs=[_row_spec(HH), _row_spec(HH), _row_spec(HH)],
        out_shape=[jax.ShapeDtypeStruct((N, HH), jnp.float32),
                   jax.ShapeDtypeStruct((N, HH), jnp.float32),
                   jax.ShapeDtypeStruct((N, HH), jnp.float32)],
    )(x0, w1_0, deg, deg)


def _kcomb_body(an, bn, agglo_ref, agghi_ref, tplo_ref, tphi_ref,
                u_ref, w1_ref, dis_ref,
                otplo_ref, otphi_ref):
    dis = dis_ref[...][:, 0:1]
    agg = jnp.concatenate([agglo_ref[...], agghi_ref[...]], axis=1)
    tp = jnp.concatenate([tplo_ref[...], tphi_ref[...]], axis=1)
    h = jnp.maximum(dis * (agg + tp) + u_ref[...], 0.0)
    t = an * h + bn * jnp.dot(h, w1_ref[...],
                              preferred_element_type=jnp.float32)
    tp_new = dis * t
    otplo_ref[...] = tp_new[:, :HH]
    otphi_ref[...] = tp_new[:, HH:]


def _kcomb(i, agglo, agghi, tplo, tphi, u_i, w1_n, dis):
    bn = _BETAS[i + 1]
    body = functools.partial(_kcomb_body, (1.0 - bn) * (1.0 - ALPHA), bn)
    return pl.pallas_call(
        body,
        grid=(GRID,),
        in_specs=[_row_spec(HH), _row_spec(HH), _row_spec(HH), _row_spec(HH),
                  _row_spec(H), _full_spec((H, H)), _row_spec(HH)],
        out_specs=[_row_spec(HH), _row_spec(HH)],
        out_shape=[jax.ShapeDtypeStruct((N, HH), jnp.float32),
                   jax.ShapeDtypeStruct((N, HH), jnp.float32)],
    )(agglo, agghi, tplo, tphi, u_i, w1_n, dis)


def _kfin_body(agglo_ref, agghi_ref, tplo_ref, tphi_ref,
               u_ref, fc1t_ref, fc1b_ref, dis_ref, y_ref):
    dis = dis_ref[...][:, 0:1]
    agg = jnp.concatenate([agglo_ref[...], agghi_ref[...]], axis=1)
    tp = jnp.concatenate([tplo_ref[...], tphi_ref[...]], axis=1)
    h = jnp.maximum(dis * (agg + tp) + u_ref[...], 0.0)
    y_ref[...] = jnp.dot(h, fc1t_ref[...],
                         preferred_element_type=jnp.float32) + fc1b_ref[...]


def _kfin(agglo, agghi, tplo, tphi, u_i, fc1_wt, fc1_b2, dis):
    return pl.pallas_call(
        _kfin_body,
        grid=(GRID,),
        in_specs=[_row_spec(HH), _row_spec(HH), _row_spec(HH), _row_spec(HH),
                  _row_spec(H), _full_spec((H, C)), _full_spec((1, C)),
                  _row_spec(HH)],
        out_specs=[_row_spec(C)],
        out_shape=[jax.ShapeDtypeStruct((N, C), jnp.float32)],
    )(agglo, agghi, tplo, tphi, u_i, fc1_wt, fc1_b2, dis)[0]


def kernel(x, edge_index, fc0_w, fc0_b, fc1_w, fc1_b, w1, w2):
    row3 = edge_index[0].reshape(NT * NPH, CPP, CHUNK)
    col3 = edge_index[1].reshape(NT * NPH, CPP, CHUNK)
    col_deg = edge_index[1].reshape(NT * 2, DCPP, DCH)

    deg = _sc_deg(col_deg)
    x0 = _k0a(x, fc0_w.T, fc0_b.reshape(1, H))

    u = [_ku(i, x0, w2[i]) for i in range(L)]

    tplo, tphi, dis = _k0b(x0, w1[0], deg)
    for i in range(L - 1):
        agglo, agghi = _sc_scatter(tplo, tphi, row3, col3)
        tplo, tphi = _kcomb(i, agglo, agghi, tplo, tphi, u[i],
                            w1[i + 1], dis)
    agglo, agghi = _sc_scatter(tplo, tphi, row3, col3)
    return _kfin(agglo, agghi, tplo, tphi, u[L - 1],
                 fc1_w.T, fc1_b.reshape(1, C), dis)
